# Initial kernel scaffold; baseline (speedup 1.0000x reference)
import jax
import jax.numpy as jnp
from jax.experimental import pallas as pl


def kernel(seq_word, seq_pos, seq_ner, word_table, pos_table, ner_table):
    raise NotImplementedError("write your pallas kernel here")


# speedup vs baseline: 1.4092x; 1.4092x over previous
import functools

import jax
import jax.numpy as jnp
from jax import lax
from jax.experimental import pallas as pl
from jax.experimental.pallas import tpu as pltpu
from jax.experimental.pallas import tpu_sc as plsc

INP_DIM = 64
TAG_DIM = 16
OUT_DIM = INP_DIM + 2 * TAG_DIM
CHUNK = 128


@functools.cache
def _build(n_rows: int):
    info = plsc.get_sparse_core_info()
    nw = info.num_cores * info.num_subcores
    assert n_rows % (nw * CHUNK) == 0
    per_w = n_rows // nw
    n_chunks = per_w // CHUNK

    mesh = plsc.VectorSubcoreMesh(core_axis_name="c", subcore_axis_name="s")

    @functools.partial(
        pl.kernel,
        mesh=mesh,
        out_type=jax.ShapeDtypeStruct((n_rows, OUT_DIM), jnp.float32),
        scratch_types=[
            pltpu.VMEM((n_chunks, CHUNK), jnp.int32),
            pltpu.VMEM((n_chunks, CHUNK), jnp.int32),
            pltpu.VMEM((n_chunks, CHUNK), jnp.int32),
            pltpu.VMEM((CHUNK, INP_DIM), jnp.float32),
            pltpu.VMEM((CHUNK, TAG_DIM), jnp.float32),
            pltpu.VMEM((CHUNK, TAG_DIM), jnp.float32),
            pltpu.SemaphoreType.DMA,
            pltpu.SemaphoreType.DMA,
            pltpu.SemaphoreType.DMA,
        ],
        compiler_params=pltpu.CompilerParams(use_tc_tiling_on_sc=False),
    )
    def k(widx_hbm, pidx_hbm, nidx_hbm, wtab_hbm, ptab_hbm, ntab_hbm,
          out_hbm, widx_v, pidx_v, nidx_v, wrows_v, prows_v, nrows_v,
          wsem, psem, nsem):
        wid = lax.axis_index("s") * info.num_cores + lax.axis_index("c")
        pltpu.sync_copy(widx_hbm.at[wid], widx_v)
        pltpu.sync_copy(pidx_hbm.at[wid], pidx_v)
        pltpu.sync_copy(nidx_hbm.at[wid], nidx_v)

        def body(c, carry):
            base = (wid * per_w + c * CHUNK).astype(jnp.int32)
            wcp = pltpu.async_copy(wtab_hbm.at[widx_v.at[c]], wrows_v, wsem)
            pcp = pltpu.async_copy(ptab_hbm.at[pidx_v.at[c]], prows_v, psem)
            ncp = pltpu.async_copy(ntab_hbm.at[nidx_v.at[c]], nrows_v, nsem)
            wcp.wait()
            pltpu.sync_copy(
                wrows_v, out_hbm.at[pl.ds(base, CHUNK), pl.ds(0, INP_DIM)])
            pcp.wait()
            pltpu.sync_copy(
                prows_v,
                out_hbm.at[pl.ds(base, CHUNK), pl.ds(INP_DIM, TAG_DIM)])
            ncp.wait()
            pltpu.sync_copy(
                nrows_v,
                out_hbm.at[pl.ds(base, CHUNK),
                           pl.ds(INP_DIM + TAG_DIM, TAG_DIM)])
            return carry

        lax.fori_loop(0, n_chunks, body, 0, unroll=False)

    def run(widx, pidx, nidx, wtab, ptab, ntab):
        shp = (nw, n_chunks, CHUNK)
        return k(widx.reshape(shp), pidx.reshape(shp), nidx.reshape(shp),
                 wtab, ptab, ntab)

    return run


def kernel(seq_word, seq_pos, seq_ner, word_table, pos_table, ner_table):
    s, b = seq_word.shape
    n = s * b
    run = _build(n)
    out = run(
        seq_word.reshape(n).astype(jnp.int32),
        seq_pos.reshape(n).astype(jnp.int32),
        seq_ner.reshape(n).astype(jnp.int32),
        word_table, pos_table, ner_table)
    return out.reshape(s, b, OUT_DIM)
